# baseline (speedup 1.0000x reference)
import jax
import jax.numpy as jnp
from jax.experimental import pallas as pl
from jax.experimental.pallas import tpu as pltpu

B, DIN = 16384, 768
H1, H2, D = 512, 256, 64
L, K = 3, 1024

BLK = 4096
SUB = 8
SB = BLK // SUB
KC = 512


def _tokenizer_kernel(x_ref, w1_ref, b1_ref, w2_ref, b2_ref, w3_ref, b3_ref,
                      cbcat_ref, cb_ref, cb2_ref,
                      ids_ref, quant_ref):
    f32 = jnp.float32
    iota = jax.lax.broadcasted_iota(jnp.int32, (SB, K), 1)
    dn = (((1,), (0,)), ((), ()))

    res = []
    for s in range(SUB):
        xb = x_ref[pl.ds(s * SB, SB), :]
        h = jnp.maximum(jnp.dot(xb, w1_ref[...], preferred_element_type=f32) + b1_ref[...], 0.0)
        h = jnp.maximum(jnp.dot(h, w2_ref[...], preferred_element_type=f32) + b2_ref[...], 0.0)
        res.append(jnp.dot(h, w3_ref[...], preferred_element_type=f32) + b3_ref[...])
    quant = [jnp.zeros((SB, D), f32) for _ in range(SUB)]
    idxs = [[] for _ in range(SUB)]
    dnt = (((1,), (1,)), ((), ()))
    for l in range(L):
        scores = [jnp.sum(res[s] * res[s], axis=-1, keepdims=True)
                  + jax.lax.dot_general(-2.0 * res[s], cb_ref[l], dnt,
                                        preferred_element_type=f32)
                  + cb2_ref[l]
                  for s in range(SUB)]
        ms = [jnp.min(sc, axis=-1, keepdims=True) for sc in scores]
        idl = [jnp.min(jnp.where(sc == m, iota, K), axis=-1, keepdims=True)
               for sc, m in zip(scores, ms)]
        onehots = [(iota == idx).astype(jnp.bfloat16) for idx in idl]
        for s in range(SUB):
            sel3 = jax.lax.dot_general(onehots[s], cbcat_ref[l], dn,
                                       preferred_element_type=f32)
            sel = (sel3[:, 0:D] + sel3[:, D:2 * D]) + sel3[:, 2 * D:3 * D]
            quant[s] = quant[s] + sel
            res[s] = res[s] - sel
            idxs[s].append(idl[s])
    for s in range(SUB):
        rows = pl.ds(s * SB, SB)
        ids_ref[rows, :] = jnp.concatenate(idxs[s], axis=1)
        quant_ref[rows, :] = quant[s]


@jax.jit
def kernel(x, W1, b1, W2, b2, W3, b3, codebooks):
    f32 = jnp.float32
    bf16 = jnp.bfloat16
    cb2 = jnp.sum(codebooks * codebooks, axis=-1)[:, None, :]
    cb_hi = jax.lax.optimization_barrier(codebooks.astype(bf16))
    r1 = codebooks - cb_hi.astype(f32)
    cb_mid = jax.lax.optimization_barrier(r1.astype(bf16))
    cb_lo = (r1 - cb_mid.astype(f32)).astype(bf16)
    cb_cat = jnp.concatenate([cb_hi, cb_mid, cb_lo], axis=-1)
    grid = (B // BLK,)
    rep = lambda *_: (0, 0)
    rep3 = lambda *_: (0, 0, 0)
    ids, quant = pl.pallas_call(
        _tokenizer_kernel,
        grid=grid,
        in_specs=[
            pl.BlockSpec((BLK, DIN), lambda i: (i, 0)),
            pl.BlockSpec((DIN, H1), rep),
            pl.BlockSpec((1, H1), rep),
            pl.BlockSpec((H1, H2), rep),
            pl.BlockSpec((1, H2), rep),
            pl.BlockSpec((H2, D), rep),
            pl.BlockSpec((1, D), rep),
            pl.BlockSpec((L, K, 3 * D), rep3),
            pl.BlockSpec((L, K, D), rep3),
            pl.BlockSpec((L, 1, K), rep3),
        ],
        out_specs=[
            pl.BlockSpec((BLK, L), lambda i: (i, 0)),
            pl.BlockSpec((BLK, D), lambda i: (i, 0)),
        ],
        out_shape=[
            jax.ShapeDtypeStruct((B, L), jnp.int32),
            jax.ShapeDtypeStruct((B, D), jnp.float32),
        ],
        compiler_params=pltpu.CompilerParams(
            dimension_semantics=("arbitrary",),
        ),
    )(x, W1, b1[None, :], W2, b2[None, :], W3, b3[None, :],
      cb_cat, codebooks, cb2)
    return ids, quant


# speedup vs baseline: 1.0236x; 1.0236x over previous
import jax
import jax.numpy as jnp
from jax.experimental import pallas as pl
from jax.experimental.pallas import tpu as pltpu

B, DIN = 16384, 768
H1, H2, D = 512, 256, 64
L, K = 3, 1024

BLK = 4096
SUB = 8
SB = BLK // SUB
KC = 256


def _tokenizer_kernel(x_ref, w1_ref, b1_ref, w2_ref, b2_ref, w3_ref, b3_ref,
                      cbcat_ref, cbt_ref, cb2_ref,
                      ids_ref, quant_ref):
    f32 = jnp.float32
    iota = jax.lax.broadcasted_iota(jnp.int32, (SB, K), 1)
    dn = (((1,), (0,)), ((), ()))

    res = []
    for s in range(SUB):
        xb = x_ref[pl.ds(s * SB, SB), :]
        h = jnp.maximum(jnp.dot(xb, w1_ref[...], preferred_element_type=f32) + b1_ref[...], 0.0)
        h = jnp.maximum(jnp.dot(h, w2_ref[...], preferred_element_type=f32) + b2_ref[...], 0.0)
        res.append(jnp.dot(h, w3_ref[...], preferred_element_type=f32) + b3_ref[...])
    quant = [jnp.zeros((SB, D), f32) for _ in range(SUB)]
    idxs = [[] for _ in range(SUB)]
    for l in range(L):
        scores = [jnp.sum(res[s] * res[s], axis=-1, keepdims=True)
                  + jnp.dot(res[s], cbt_ref[l], preferred_element_type=f32)
                  + cb2_ref[l]
                  for s in range(SUB)]
        ms = [jnp.min(sc, axis=-1, keepdims=True) for sc in scores]
        idl = [jnp.min(jnp.where(sc == m, iota, K), axis=-1, keepdims=True)
               for sc, m in zip(scores, ms)]
        onehots = [(iota == idx).astype(jnp.bfloat16) for idx in idl]
        for s in range(SUB):
            sel3 = jax.lax.dot_general(onehots[s], cbcat_ref[l], dn,
                                       preferred_element_type=f32)
            sel = (sel3[:, 0:D] + sel3[:, D:2 * D]) + sel3[:, 2 * D:3 * D]
            quant[s] = quant[s] + sel
            res[s] = res[s] - sel
            idxs[s].append(idl[s])
    for s in range(SUB):
        rows = pl.ds(s * SB, SB)
        ids_ref[rows, :] = jnp.concatenate(idxs[s], axis=1)
        quant_ref[rows, :] = quant[s]


@jax.jit
def kernel(x, W1, b1, W2, b2, W3, b3, codebooks):
    f32 = jnp.float32
    bf16 = jnp.bfloat16
    cbt = jnp.transpose(-2.0 * codebooks, (0, 2, 1))
    cb2 = jnp.sum(codebooks * codebooks, axis=-1)[:, None, :]
    cb_hi = jax.lax.optimization_barrier(codebooks.astype(bf16))
    r1 = codebooks - cb_hi.astype(f32)
    cb_mid = jax.lax.optimization_barrier(r1.astype(bf16))
    cb_lo = (r1 - cb_mid.astype(f32)).astype(bf16)
    cb_cat = jnp.concatenate([cb_hi, cb_mid, cb_lo], axis=-1)
    grid = (B // BLK,)
    rep = lambda *_: (0, 0)
    rep3 = lambda *_: (0, 0, 0)
    ids, quant = pl.pallas_call(
        _tokenizer_kernel,
        grid=grid,
        in_specs=[
            pl.BlockSpec((BLK, DIN), lambda i: (i, 0)),
            pl.BlockSpec((DIN, H1), rep),
            pl.BlockSpec((1, H1), rep),
            pl.BlockSpec((H1, H2), rep),
            pl.BlockSpec((1, H2), rep),
            pl.BlockSpec((H2, D), rep),
            pl.BlockSpec((1, D), rep),
            pl.BlockSpec((L, K, 3 * D), rep3),
            pl.BlockSpec((L, D, K), rep3),
            pl.BlockSpec((L, 1, K), rep3),
        ],
        out_specs=[
            pl.BlockSpec((BLK, L), lambda i: (i, 0)),
            pl.BlockSpec((BLK, D), lambda i: (i, 0)),
        ],
        out_shape=[
            jax.ShapeDtypeStruct((B, L), jnp.int32),
            jax.ShapeDtypeStruct((B, D), jnp.float32),
        ],
        compiler_params=pltpu.CompilerParams(
            dimension_semantics=("parallel",),
        ),
    )(x, W1, b1[None, :], W2, b2[None, :], W3, b3[None, :],
      cb_cat, cbt, cb2)
    return ids, quant


# speedup vs baseline: 1.0330x; 1.0092x over previous
import jax
import jax.numpy as jnp
from jax.experimental import pallas as pl
from jax.experimental.pallas import tpu as pltpu

B, DIN = 16384, 768
H1, H2, D = 512, 256, 64
L, K = 3, 1024

BLK = 4096
SUB = 8
SB = BLK // SUB


def _tokenizer_kernel(x_ref, w1_ref, b1_ref, w2_ref, b2_ref, w3_ref, b3_ref,
                      cb_ref, ids_ref, quant_ref,
                      cbt_ref, cbcat_ref, cb2_ref):
    f32 = jnp.float32
    bf16 = jnp.bfloat16
    iota = jax.lax.broadcasted_iota(jnp.int32, (SB, K), 1)
    dn = (((1,), (0,)), ((), ()))

    @pl.when(pl.program_id(0) == 0)
    def _prep():
        for l in range(L):
            cb = cb_ref[l]
            cbt_ref[l] = jnp.transpose(-2.0 * cb)
            cb2col = jnp.sum(cb * cb, axis=-1, keepdims=True)
            cb2_ref[l] = jnp.transpose(cb2col)
            cbcat_ref[l, :, 0:D] = cb.astype(bf16)
            r1 = cb - cbcat_ref[l, :, 0:D].astype(f32)
            cbcat_ref[l, :, D:2 * D] = r1.astype(bf16)
            cbcat_ref[l, :, 2 * D:3 * D] = (
                r1 - cbcat_ref[l, :, D:2 * D].astype(f32)).astype(bf16)

    res = []
    for s in range(SUB):
        xb = x_ref[pl.ds(s * SB, SB), :]
        h = jnp.maximum(jnp.dot(xb, w1_ref[...], preferred_element_type=f32) + b1_ref[...], 0.0)
        h = jnp.maximum(jnp.dot(h, w2_ref[...], preferred_element_type=f32) + b2_ref[...], 0.0)
        res.append(jnp.dot(h, w3_ref[...], preferred_element_type=f32) + b3_ref[...])
    quant = [jnp.zeros((SB, D), f32) for _ in range(SUB)]
    idxs = [[] for _ in range(SUB)]
    for l in range(L):
        scores = [jnp.sum(res[s] * res[s], axis=-1, keepdims=True)
                  + jnp.dot(res[s], cbt_ref[l], preferred_element_type=f32)
                  + cb2_ref[l]
                  for s in range(SUB)]
        ms = [jnp.min(sc, axis=-1, keepdims=True) for sc in scores]
        idl = [jnp.min(jnp.where(sc == m, iota, K), axis=-1, keepdims=True)
               for sc, m in zip(scores, ms)]
        onehots = [(iota == idx).astype(bf16) for idx in idl]
        for s in range(SUB):
            sel3 = jax.lax.dot_general(onehots[s], cbcat_ref[l], dn,
                                       preferred_element_type=f32)
            sel = (sel3[:, 0:D] + sel3[:, D:2 * D]) + sel3[:, 2 * D:3 * D]
            quant[s] = quant[s] + sel
            res[s] = res[s] - sel
            idxs[s].append(idl[s])
    for s in range(SUB):
        rows = pl.ds(s * SB, SB)
        ids_ref[rows, :] = jnp.concatenate(idxs[s], axis=1)
        quant_ref[rows, :] = quant[s]


@jax.jit
def kernel(x, W1, b1, W2, b2, W3, b3, codebooks):
    grid = (B // BLK,)
    rep = lambda *_: (0, 0)
    rep3 = lambda *_: (0, 0, 0)
    ids, quant = pl.pallas_call(
        _tokenizer_kernel,
        grid=grid,
        in_specs=[
            pl.BlockSpec((BLK, DIN), lambda i: (i, 0)),
            pl.BlockSpec((DIN, H1), rep),
            pl.BlockSpec((1, H1), rep),
            pl.BlockSpec((H1, H2), rep),
            pl.BlockSpec((1, H2), rep),
            pl.BlockSpec((H2, D), rep),
            pl.BlockSpec((1, D), rep),
            pl.BlockSpec((L, K, D), rep3),
        ],
        out_specs=[
            pl.BlockSpec((BLK, L), lambda i: (i, 0)),
            pl.BlockSpec((BLK, D), lambda i: (i, 0)),
        ],
        out_shape=[
            jax.ShapeDtypeStruct((B, L), jnp.int32),
            jax.ShapeDtypeStruct((B, D), jnp.float32),
        ],
        scratch_shapes=[
            pltpu.VMEM((L, D, K), jnp.float32),
            pltpu.VMEM((L, K, 3 * D), jnp.bfloat16),
            pltpu.VMEM((L, 1, K), jnp.float32),
        ],
        compiler_params=pltpu.CompilerParams(
            dimension_semantics=("arbitrary",),
        ),
    )(x, W1, b1[None, :], W2, b2[None, :], W3, b3[None, :], codebooks)
    return ids, quant
